# baseline (speedup 1.0000x reference)
import functools
import jax
import jax.numpy as jnp
from jax import lax
from jax.experimental import pallas as pl
from jax.experimental.pallas import tpu as pltpu
from jax.experimental.pallas import tpu_sc as plsc

NCH = 32

U = 3
D_P = 16
TOPK = 32
D = 512
N_H = 2
M = D_P ** U
B = 16
T = 4
NROWS = 2 * B * T
NFILT = 40
BIGI = 1 << 30


def _pre_body(p_ref, h_ref, hhk_ref, hhvk_ref, hhvv_ref, hhbw_ref, hhbb_ref,
              gamma_ref, wgate_ref,
              oidx_ref, oval_ref, decay_ref, khat_ref, beta_ref,
              addk_ref, addv_ref, gate_ref,
              sval_ref, sidx_ref):
    p = p_ref[...]
    p0 = p[:, 0, :]
    p1 = p[:, 1, :]
    p2 = p[:, 2, :]
    a2 = (p0[:, :, None] * p1[:, None, :]).reshape(NROWS, 256)
    iota256 = lax.broadcasted_iota(jnp.int32, (NROWS, 256), 1)
    a2m = a2
    for i in range(NFILT):
        m = jnp.max(a2m, axis=1, keepdims=True)
        am = jnp.min(jnp.where(a2m == m, iota256, BIGI), axis=1, keepdims=True)
        sval_ref[:, i:i + 1] = m
        sidx_ref[:, i:i + 1] = am
        a2m = jnp.where(iota256 == am, -jnp.inf, a2m)
    cval = sval_ref[...]
    cidx = sidx_ref[...]
    iota16 = lax.broadcasted_iota(jnp.int32, (NROWS, NFILT, 16), 2)
    c3 = (cval[:, :, None] * p2[:, None, :]).reshape(NROWS, NFILT * 16)
    g3 = (cidx[:, :, None] * 16 + iota16).reshape(NROWS, NFILT * 16)
    for i in range(TOPK):
        m = jnp.max(c3, axis=1, keepdims=True)
        g = jnp.min(jnp.where(c3 == m, g3, BIGI), axis=1, keepdims=True)
        oval_ref[:, i:i + 1] = m
        oidx_ref[:, i:i + 1] = g
        c3 = jnp.where((c3 == m) & (g3 == g), -jnp.inf, c3)

    gamma = gamma_ref[0, 0]
    sp = jnp.log(1.0 + jnp.exp(gamma))
    wv = oval_ref[0:B * T, :]
    decay_ref[...] = jnp.exp(sp * jnp.log(jnp.maximum(1.0 - wv, 0.0)))

    h2 = h_ref[...]
    dn = (((1,), (1,)), ((), ()))
    for j in range(N_H):
        kj_lin = lax.dot_general(h2, hhk_ref[j], dn)
        kj = kj_lin * (1.0 / (1.0 + jnp.exp(-kj_lin)))
        nrm = jnp.maximum(
            jnp.sqrt(jnp.sum(kj * kj, axis=1, keepdims=True)), 1e-12)
        khat_ref[j] = kj / nrm
        vkj = lax.dot_general(h2, hhvk_ref[j], dn)
        vvj = lax.dot_general(h2, hhvv_ref[j], dn)
        blin = jnp.sum(h2 * hhbw_ref[j:j + 1, :], axis=1, keepdims=True)
        betaj = 2.0 / (1.0 + jnp.exp(-(blin + hhbb_ref[0, j])))
        beta_ref[:, j:j + 1] = betaj
        addk_ref[j] = (betaj * kj) * vkj
        addv_ref[j] = (betaj * kj) * vvj
    gw = lax.dot_general(h2, wgate_ref[...], dn)
    gate_ref[...] = 1.0 / (1.0 + jnp.exp(-gw))


def _run_pre(probs, h2, hh_k_w, hh_vk_w, hh_vv_w, hh_beta_w, hh_beta_b,
             gamma, W_gate):
    f32 = jnp.float32
    outs = [
        jax.ShapeDtypeStruct((NROWS, TOPK), jnp.int32),
        jax.ShapeDtypeStruct((NROWS, TOPK), f32),
        jax.ShapeDtypeStruct((B * T, TOPK), f32),
        jax.ShapeDtypeStruct((N_H, B * T, D), f32),
        jax.ShapeDtypeStruct((B * T, N_H), f32),
        jax.ShapeDtypeStruct((N_H, B * T, D), f32),
        jax.ShapeDtypeStruct((N_H, B * T, D), f32),
        jax.ShapeDtypeStruct((B * T, D), f32),
    ]
    return pl.pallas_call(
        _pre_body,
        out_shape=outs,
        scratch_shapes=[
            pltpu.VMEM((NROWS, NFILT), f32),
            pltpu.VMEM((NROWS, NFILT), jnp.int32),
        ],
    )(probs, h2, hh_k_w, hh_vk_w, hh_vv_w, hh_beta_w, hh_beta_b,
      gamma.reshape(1, 1), W_gate)


def _state_body(*refs):
    (Kf, Vf, zKi, zVi, widx_h, ridx_h, dec_h, wv_h, khat_h, addk_h, addv_h,
     beta_h, h_h,
     ro_o, zKo, zVo, cK_o, cV_o,
     pos, zk_l, zv_l, widx_l, ridx_l, dec_s, wv_s, khat_l, addk_l, addv_l,
     beta_l, h_l, rK, rV, cKb, cVb, sidx, srel, sorel, szk, szv, swv,
     part, tmp512, relX, partX, semA, semB, semC, semD) = refs
    f32 = jnp.float32
    i32 = jnp.int32
    c = lax.axis_index("c")
    s = lax.axis_index("s")
    batch = c * 8 + s // 2
    half = s % 2
    lb = s // 2
    scale = float(D) ** (-0.5)

    pltpu.sync_copy(widx_h.at[batch], widx_l)
    pltpu.sync_copy(ridx_h.at[batch], ridx_l)
    pltpu.sync_copy(dec_h.at[batch], dec_s.at[pl.ds(0, 128)])
    pltpu.sync_copy(wv_h.at[batch], wv_s)
    pltpu.sync_copy(khat_h.at[batch], khat_l)
    pltpu.sync_copy(addk_h.at[batch], addk_l)
    pltpu.sync_copy(addv_h.at[batch], addv_l)
    pltpu.sync_copy(beta_h.at[batch], beta_l.at[pl.ds(0, 128)])
    pltpu.sync_copy(h_h.at[batch], h_l)
    pltpu.sync_copy(zKi.at[batch], zk_l)
    pltpu.sync_copy(zVi.at[batch], zv_l)

    def ini(i, carry):
        pos[pl.ds(i * 16, 16)] = jnp.full((16,), -1, i32)
        return carry
    lax.fori_loop(0, M // 16, ini, 0)

    ii16 = lax.iota(i32, 16)

    def step(t, carry):
        base = t * 32
        my = base + half * 16
        wi = widx_l[pl.ds(my, 16)]
        slots = plsc.load_gather(pos, [wi])
        gidx = batch * M + wi
        cidx = batch * 128 + jnp.maximum(slots, 0)
        cp1 = pltpu.async_copy(Kf.at[gidx], rK, semA)
        cp2 = pltpu.async_copy(Vf.at[gidx], rV, semB)
        cp3 = pltpu.async_copy(cK_o.at[cidx], cKb, semC)
        cp4 = pltpu.async_copy(cV_o.at[cidx], cVb, semD)
        cp1.wait()
        cp2.wait()
        cp3.wait()
        cp4.wait()
        sidx[pl.ds(0, 16)] = slots

        def row_fn(i2, carry2):
          for sub in range(2):
            i = i2 * 2 + sub
            sl = sidx[pl.ds(i, 16)][0]

            @pl.when(sl >= 0)
            def _():
                for cc in range(NCH):
                    rK[i, pl.ds(cc * 16, 16)] = cKb[i, pl.ds(cc * 16, 16)]
                    rV[i, pl.ds(cc * 16, 16)] = cVb[i, pl.ds(cc * 16, 16)]

            dec = dec_s[pl.ds(my + i, 16)][0]
            for j in range(N_H):
                bj = beta_l[pl.ds(t * 2 + j, 16)][0]
                kb = (t * 2 + j) * D
                a0k = jnp.zeros((16,), f32)
                a1k = jnp.zeros((16,), f32)
                a0v = jnp.zeros((16,), f32)
                a1v = jnp.zeros((16,), f32)
                for cc in range(0, NCH, 2):
                    kh0 = khat_l[pl.ds(kb + cc * 16, 16)]
                    kh1 = khat_l[pl.ds(kb + cc * 16 + 16, 16)]
                    a0k = a0k + kh0 * rK[i, pl.ds(cc * 16, 16)]
                    a1k = a1k + kh1 * rK[i, pl.ds(cc * 16 + 16, 16)]
                    a0v = a0v + kh0 * rV[i, pl.ds(cc * 16, 16)]
                    a1v = a1v + kh1 * rV[i, pl.ds(cc * 16 + 16, 16)]
                sk = jnp.sum(a0k + a1k)
                sv = jnp.sum(a0v + a1v)
                if j == 0:
                    dk = bj * (dec * sk)
                    dv = bj * (dec * sv)
                else:
                    dk = bj * sk
                    dv = bj * sv
                for cc in range(NCH):
                    kh = khat_l[pl.ds(kb + cc * 16, 16)]
                    ak = addk_l[pl.ds(kb + cc * 16, 16)]
                    av = addv_l[pl.ds(kb + cc * 16, 16)]
                    if j == 0:
                        rK[i, pl.ds(cc * 16, 16)] = (
                            rK[i, pl.ds(cc * 16, 16)] * dec - kh * dk + ak)
                        rV[i, pl.ds(cc * 16, 16)] = (
                            rV[i, pl.ds(cc * 16, 16)] * dec - kh * dv + av)
                    else:
                        rK[i, pl.ds(cc * 16, 16)] = (
                            rK[i, pl.ds(cc * 16, 16)] - kh * dk + ak)
                        rV[i, pl.ds(cc * 16, 16)] = (
                            rV[i, pl.ds(cc * 16, 16)] - kh * dv + av)
          return carry2
        lax.fori_loop(0, 8, row_fn, 0)
        pltpu.sync_copy(rK, cK_o.at[pl.ds(batch * 128 + my, 16)])
        pltpu.sync_copy(rV, cV_o.at[pl.ds(batch * 128 + my, 16)])
        for vh in range(2):
            wiv = widx_l[pl.ds(base + vh * 16, 16)]
            plsc.store_scatter(pos, [wiv], base + vh * 16 + ii16)
            dcv = dec_s[pl.ds(base + vh * 16, 16)]
            wvv = wv_s[pl.ds(base + vh * 16, 16)]
            zk = plsc.load_gather(zk_l, [wiv])
            plsc.store_scatter(zk_l, [wiv], dcv * zk + wvv)
            zv = plsc.load_gather(zv_l, [wiv])
            plsc.store_scatter(zv_l, [wiv], dcv * zv + wvv)
        plsc.subcore_barrier()
        ri = ridx_l[pl.ds(my, 16)]
        rsl = plsc.load_gather(pos, [ri])
        gidx2 = batch * M + ri
        cidx2 = batch * 128 + jnp.maximum(rsl, 0)
        cp5 = pltpu.async_copy(Kf.at[gidx2], rK, semA)
        cp6 = pltpu.async_copy(Vf.at[gidx2], rV, semB)
        cp7 = pltpu.async_copy(cK_o.at[cidx2], cKb, semC)
        cp8 = pltpu.async_copy(cV_o.at[cidx2], cVb, semD)
        cp5.wait()
        cp6.wait()
        cp7.wait()
        cp8.wait()
        sidx[pl.ds(0, 16)] = rsl

        def sel_fn(i, carry2):
            sl = sidx[pl.ds(i, 16)][0]

            @pl.when(sl >= 0)
            def _():
                for cc in range(NCH):
                    rK[i, pl.ds(cc * 16, 16)] = cKb[i, pl.ds(cc * 16, 16)]
                    rV[i, pl.ds(cc * 16, 16)] = cVb[i, pl.ds(cc * 16, 16)]
            return carry2
        lax.fori_loop(0, 16, sel_fn, 0)
        szk[...] = plsc.load_gather(zk_l, [ri])
        szv[...] = plsc.load_gather(zv_l, [ri])

        def rel_fn(i, carry2):
            acc = jnp.zeros((16,), f32)
            for cc in range(NCH):
                acc = acc + (rK[i, pl.ds(cc * 16, 16)]
                             * h_l[pl.ds(t * D + cc * 16, 16)])
            return jnp.where(ii16 == i, jnp.sum(acc), carry2)
        relsum = lax.fori_loop(0, 16, rel_fn, jnp.zeros((16,), f32))
        srel[...] = relsum / szk[...] * scale
        pltpu.sync_copy(srel, relX.at[lb, half])
        plsc.subcore_barrier()
        pltpu.sync_copy(relX.at[lb, 1 - half], sorel)
        a = srel[...]
        bvec = sorel[...]
        m = jnp.maximum(jnp.max(a), jnp.max(bvec))
        ea = jnp.exp(a - m)
        eb = jnp.exp(bvec - m)
        ssum = jnp.sum(ea) + jnp.sum(eb)
        swv[...] = (ea / ssum) / szv[...]

        def ch_fn(cc, carry2):
            wv = swv[...]
            acc = jnp.zeros((16,), f32)
            for i in range(16):
                acc = acc + wv[i] * rV[i, pl.ds(cc * 16, 16)]
            part[pl.ds(cc * 16, 16)] = acc
            return carry2
        lax.fori_loop(0, NCH, ch_fn, 0)
        pltpu.sync_copy(part, partX.at[lb, half])
        plsc.subcore_barrier()

        @pl.when(half == 0)
        def _():
            pltpu.sync_copy(partX.at[lb, 1], tmp512)
            for cc in range(NCH):
                part[pl.ds(cc * 16, 16)] = (part[pl.ds(cc * 16, 16)]
                                            + tmp512[pl.ds(cc * 16, 16)])
            pltpu.sync_copy(part, ro_o.at[batch, pl.ds(t * D, D)])
        return carry
    lax.fori_loop(0, T, step, 0)

    @pl.when(half == 0)
    def _():
        pltpu.sync_copy(zk_l, zKo.at[batch])
        pltpu.sync_copy(zv_l, zVo.at[batch])


def _run_state(Kf, Vf, zK, zV, widx_f, ridx_f, dec_f, wv_f, khat_f,
               addk_f, addv_f, beta_f, h_f):
    f32 = jnp.float32
    i32 = jnp.int32
    mesh = plsc.VectorSubcoreMesh(core_axis_name="c", subcore_axis_name="s")
    outs = (jax.ShapeDtypeStruct((B, T * D), f32),
            jax.ShapeDtypeStruct((B, M), f32),
            jax.ShapeDtypeStruct((B, M), f32),
            jax.ShapeDtypeStruct((B * 128, D), f32),
            jax.ShapeDtypeStruct((B * 128, D), f32))
    scratch = [
        pltpu.VMEM((M,), i32),
        pltpu.VMEM((M,), f32),
        pltpu.VMEM((M,), f32),
        pltpu.VMEM((128,), i32),
        pltpu.VMEM((128,), i32),
        pltpu.VMEM((160,), f32),
        pltpu.VMEM((128,), f32),
        pltpu.VMEM((T * N_H * D,), f32),
        pltpu.VMEM((T * N_H * D,), f32),
        pltpu.VMEM((T * N_H * D,), f32),
        pltpu.VMEM((144,), f32),
        pltpu.VMEM((T * D,), f32),
        pltpu.VMEM((16, D), f32),
        pltpu.VMEM((16, D), f32),
        pltpu.VMEM((16, D), f32),
        pltpu.VMEM((16, D), f32),
        pltpu.VMEM((32,), i32),
        pltpu.VMEM((16,), f32),
        pltpu.VMEM((16,), f32),
        pltpu.VMEM((16,), f32),
        pltpu.VMEM((16,), f32),
        pltpu.VMEM((16,), f32),
        pltpu.VMEM((D,), f32),
        pltpu.VMEM((D,), f32),
        pltpu.VMEM_SHARED((8, 2, 16), f32),
        pltpu.VMEM_SHARED((8, 2, D), f32),
        pltpu.SemaphoreType.DMA,
        pltpu.SemaphoreType.DMA,
        pltpu.SemaphoreType.DMA,
        pltpu.SemaphoreType.DMA,
    ]
    fn = pl.kernel(_state_body, out_type=outs, mesh=mesh,
                   scratch_types=scratch,
                   compiler_params=pltpu.CompilerParams(
                       needs_layout_passes=False))
    return fn(Kf, Vf, zK, zV, widx_f, ridx_f, dec_f, wv_f, khat_f,
              addk_f, addv_f, beta_f, h_f)


BLKM = 2048
NBLK = M // BLKM


def _merge_body(keys_ref, kin, vin, ck, cv, kout, vout, p_ref):
    b = pl.program_id(0)
    n = pl.program_id(1)
    kout[...] = kin[...]
    vout[...] = vin[...]

    @pl.when(n == 0)
    def _():
        p_ref[0] = 0

    lim = (n + 1) * BLKM * 128

    def cond(p):
        return jnp.logical_and(p < 128, keys_ref[b, p] < lim)

    def body(p):
        key = keys_ref[b, p]
        idx = key >> 7
        slot = key & 127
        row = idx - n * BLKM
        kout[0, pl.ds(row, 1), :] = ck[0, pl.ds(slot, 1), :]
        vout[0, pl.ds(row, 1), :] = cv[0, pl.ds(slot, 1), :]
        return p + 1
    p_ref[0] = lax.while_loop(cond, body, p_ref[0])


def _run_merge(keys, K_slots, V_slots, cacheK, cacheV):
    f32 = jnp.float32
    grid_spec = pltpu.PrefetchScalarGridSpec(
        num_scalar_prefetch=1,
        grid=(B, NBLK),
        in_specs=[
            pl.BlockSpec((1, BLKM, D), lambda b, n, keys: (b, n, 0)),
            pl.BlockSpec((1, BLKM, D), lambda b, n, keys: (b, n, 0)),
            pl.BlockSpec((1, 128, D), lambda b, n, keys: (b, 0, 0)),
            pl.BlockSpec((1, 128, D), lambda b, n, keys: (b, 0, 0)),
        ],
        out_specs=[
            pl.BlockSpec((1, BLKM, D), lambda b, n, keys: (b, n, 0)),
            pl.BlockSpec((1, BLKM, D), lambda b, n, keys: (b, n, 0)),
        ],
        scratch_shapes=[pltpu.SMEM((1,), jnp.int32)],
    )
    return pl.pallas_call(
        _merge_body,
        grid_spec=grid_spec,
        out_shape=[jax.ShapeDtypeStruct((B, M, D), f32),
                   jax.ShapeDtypeStruct((B, M, D), f32)],
    )(keys, K_slots, V_slots, cacheK, cacheV)


def _post_body(ro_ref, gate_ref, wout_ref, y_ref):
    dn = (((1,), (1,)), ((), ()))
    y_ref[...] = gate_ref[...] * lax.dot_general(ro_ref[...], wout_ref[...], dn)


def _run_post(readout2, gate, W_out):
    return pl.pallas_call(
        _post_body,
        out_shape=jax.ShapeDtypeStruct((B * T, D), jnp.float32),
    )(readout2, gate, W_out)


def kernel(h, K_slots, V_slots, z_K, z_V, W_k, W_q, log_tau_w, log_tau_r,
           hh_k_w, hh_vk_w, hh_vv_w, hh_beta_w, hh_beta_b, gamma,
           W_out, W_gate):
    tau_w = jnp.exp(log_tau_w)
    tau_r = jnp.exp(log_tau_r)
    probs_list = []
    for t in range(T):
        h_t = h[:, t]
        zw = (h_t @ W_k.T).reshape(B, U, D_P)
        zr = (h_t @ W_q.T).reshape(B, U, D_P)
        probs_list.append(jax.nn.softmax(zw / tau_w, axis=-1))
        probs_list.append(jax.nn.softmax(zr / tau_r, axis=-1))
    pw = jnp.stack(probs_list[0::2], axis=1)
    pr = jnp.stack(probs_list[1::2], axis=1)
    probs = jnp.concatenate(
        [pw.reshape(B * T, U, D_P), pr.reshape(B * T, U, D_P)], axis=0)

    h2 = h.reshape(B * T, D)
    (oidx, oval, decay, khat, beta, addk, addv, gate) = _run_pre(
        probs, h2, hh_k_w, hh_vk_w, hh_vv_w,
        hh_beta_w.reshape(N_H, D), hh_beta_b.reshape(1, N_H),
        gamma, W_gate)

    widx = oidx[:B * T].reshape(B, T, TOPK)
    ridx = oidx[B * T:].reshape(B, T, TOPK)
    wval = oval[:B * T].reshape(B, T, TOPK)
    decay = decay.reshape(B, T, TOPK)
    khat = khat.reshape(N_H, B, T, D)
    beta = beta.reshape(B, T, N_H)
    addk = addk.reshape(N_H, B, T, D)
    addv = addv.reshape(N_H, B, T, D)

    i32 = jnp.int32
    widx_f = widx.reshape(B, T * TOPK)
    ridx_f = ridx.reshape(B, T * TOPK)
    dec_f = decay.reshape(B, T * TOPK)
    wv_f = wval.reshape(B, T * TOPK)
    khat_f = khat.transpose(1, 2, 0, 3).reshape(B, T * N_H * D)
    addk_f = addk.transpose(1, 2, 0, 3).reshape(B, T * N_H * D)
    addv_f = addv.transpose(1, 2, 0, 3).reshape(B, T * N_H * D)
    beta_f = jnp.concatenate(
        [beta.reshape(B, T * N_H), jnp.zeros((B, 120), jnp.float32)], axis=1)
    h_f = h.reshape(B, T * D)
    ro, zKo, zVo, cKf, cVf = _run_state(
        K_slots.reshape(B * M, D), V_slots.reshape(B * M, D), z_K, z_V,
        widx_f, ridx_f, dec_f, wv_f, khat_f, addk_f, addv_f, beta_f, h_f)

    keys = jnp.sort(widx_f * 128
                    + jnp.arange(T * TOPK, dtype=i32)[None, :], axis=1)
    K_out, V_out = _run_merge(keys, K_slots, V_slots,
                              cKf.reshape(B, 128, D), cVf.reshape(B, 128, D))

    y = _run_post(ro.reshape(B * T, D), gate, W_out).reshape(B, T, D)
    return (y, K_out, V_out, zKo, zVo, widx[:, T - 1], ridx[:, T - 1])


# speedup vs baseline: 1.1837x; 1.1837x over previous
import functools
import jax
import jax.numpy as jnp
from jax import lax
from jax.experimental import pallas as pl
from jax.experimental.pallas import tpu as pltpu
from jax.experimental.pallas import tpu_sc as plsc

NCH = 32

U = 3
D_P = 16
TOPK = 32
D = 512
N_H = 2
M = D_P ** U
B = 16
T = 4
NROWS = 2 * B * T
NFILT = 40
BIGI = 1 << 30


def _pre_body(p_ref, h_ref, hhk_ref, hhvk_ref, hhvv_ref, hhbw_ref, hhbb_ref,
              gamma_ref, wgate_ref,
              oidx_ref, oval_ref, decay_ref, khat_ref, beta_ref,
              addk_ref, addv_ref, gate_ref,
              sval_ref, sidx_ref):
    p = p_ref[...]
    p0 = p[:, 0, :]
    p1 = p[:, 1, :]
    p2 = p[:, 2, :]
    a2 = (p0[:, :, None] * p1[:, None, :]).reshape(NROWS, 256)
    iota256 = lax.broadcasted_iota(jnp.int32, (NROWS, 256), 1)
    a2m = a2
    for i in range(NFILT):
        m = jnp.max(a2m, axis=1, keepdims=True)
        am = jnp.min(jnp.where(a2m == m, iota256, BIGI), axis=1, keepdims=True)
        sval_ref[:, i:i + 1] = m
        sidx_ref[:, i:i + 1] = am
        a2m = jnp.where(iota256 == am, -jnp.inf, a2m)
    cval = sval_ref[...]
    cidx = sidx_ref[...]
    iota16 = lax.broadcasted_iota(jnp.int32, (NROWS, NFILT, 16), 2)
    c3 = (cval[:, :, None] * p2[:, None, :]).reshape(NROWS, NFILT * 16)
    g3 = (cidx[:, :, None] * 16 + iota16).reshape(NROWS, NFILT * 16)
    for i in range(TOPK):
        m = jnp.max(c3, axis=1, keepdims=True)
        g = jnp.min(jnp.where(c3 == m, g3, BIGI), axis=1, keepdims=True)
        oval_ref[:, i:i + 1] = m
        oidx_ref[:, i:i + 1] = g
        c3 = jnp.where((c3 == m) & (g3 == g), -jnp.inf, c3)

    gamma = gamma_ref[0, 0]
    sp = jnp.log(1.0 + jnp.exp(gamma))
    wv = oval_ref[0:B * T, :]
    decay_ref[...] = jnp.exp(sp * jnp.log(jnp.maximum(1.0 - wv, 0.0)))

    h2 = h_ref[...]
    dn = (((1,), (1,)), ((), ()))
    for j in range(N_H):
        kj_lin = lax.dot_general(h2, hhk_ref[j], dn)
        kj = kj_lin * (1.0 / (1.0 + jnp.exp(-kj_lin)))
        nrm = jnp.maximum(
            jnp.sqrt(jnp.sum(kj * kj, axis=1, keepdims=True)), 1e-12)
        khat_ref[j] = kj / nrm
        vkj = lax.dot_general(h2, hhvk_ref[j], dn)
        vvj = lax.dot_general(h2, hhvv_ref[j], dn)
        blin = jnp.sum(h2 * hhbw_ref[j:j + 1, :], axis=1, keepdims=True)
        betaj = 2.0 / (1.0 + jnp.exp(-(blin + hhbb_ref[0, j])))
        beta_ref[:, j:j + 1] = betaj
        addk_ref[j] = (betaj * kj) * vkj
        addv_ref[j] = (betaj * kj) * vvj
    gw = lax.dot_general(h2, wgate_ref[...], dn)
    gate_ref[...] = 1.0 / (1.0 + jnp.exp(-gw))


def _run_pre(probs, h2, hh_k_w, hh_vk_w, hh_vv_w, hh_beta_w, hh_beta_b,
             gamma, W_gate):
    f32 = jnp.float32
    outs = [
        jax.ShapeDtypeStruct((NROWS, TOPK), jnp.int32),
        jax.ShapeDtypeStruct((NROWS, TOPK), f32),
        jax.ShapeDtypeStruct((B * T, TOPK), f32),
        jax.ShapeDtypeStruct((N_H, B * T, D), f32),
        jax.ShapeDtypeStruct((B * T, N_H), f32),
        jax.ShapeDtypeStruct((N_H, B * T, D), f32),
        jax.ShapeDtypeStruct((N_H, B * T, D), f32),
        jax.ShapeDtypeStruct((B * T, D), f32),
    ]
    return pl.pallas_call(
        _pre_body,
        out_shape=outs,
        scratch_shapes=[
            pltpu.VMEM((NROWS, NFILT), f32),
            pltpu.VMEM((NROWS, NFILT), jnp.int32),
        ],
    )(probs, h2, hh_k_w, hh_vk_w, hh_vv_w, hh_beta_w, hh_beta_b,
      gamma.reshape(1, 1), W_gate)


def _state_body(*refs):
    (Kf, Vf, zKi, zVi, widx_h, ridx_h, dec_h, wv_h, khat_h, addk_h, addv_h,
     beta_h, h_h,
     ro_o, zKo, zVo, cK_o, cV_o,
     pos, zk_l, zv_l, widx_l, ridx_l, dec_s, wv_s, khat_l, addk_l, addv_l,
     beta_l, h_l, rK, rV, cKb, cVb, sidx, srel, sorel, szk, szv, swv,
     part, tmp512, relX, partX, semA, semB, semC, semD) = refs
    f32 = jnp.float32
    i32 = jnp.int32
    c = lax.axis_index("c")
    s = lax.axis_index("s")
    batch = c * 8 + s // 2
    half = s % 2
    lb = s // 2
    scale = float(D) ** (-0.5)

    pltpu.sync_copy(widx_h.at[batch], widx_l)
    pltpu.sync_copy(ridx_h.at[batch], ridx_l)
    pltpu.sync_copy(dec_h.at[batch], dec_s.at[pl.ds(0, 128)])
    pltpu.sync_copy(wv_h.at[batch], wv_s)
    pltpu.sync_copy(khat_h.at[batch], khat_l)
    pltpu.sync_copy(addk_h.at[batch], addk_l)
    pltpu.sync_copy(addv_h.at[batch], addv_l)
    pltpu.sync_copy(beta_h.at[batch], beta_l.at[pl.ds(0, 128)])
    pltpu.sync_copy(h_h.at[batch], h_l)
    pltpu.sync_copy(zKi.at[batch], zk_l)
    pltpu.sync_copy(zVi.at[batch], zv_l)

    def ini(i, carry):
        pos[pl.ds(i * 16, 16)] = jnp.full((16,), -1, i32)
        return carry
    lax.fori_loop(0, M // 16, ini, 0)

    ii16 = lax.iota(i32, 16)

    def step(t, carry):
        base = t * 32
        my = base + half * 16
        wi = widx_l[pl.ds(my, 16)]
        slots = plsc.load_gather(pos, [wi])
        gidx = batch * M + wi
        cidx = batch * 128 + jnp.maximum(slots, 0)
        ncached = plsc.all_reduce_population_count(slots >= 0)[0]
        cp1 = pltpu.async_copy(Kf.at[gidx], rK, semA)
        cp2 = pltpu.async_copy(Vf.at[gidx], rV, semB)

        @pl.when(ncached > 0)
        def _():
            cp3 = pltpu.async_copy(cK_o.at[cidx], cKb, semC)
            cp4 = pltpu.async_copy(cV_o.at[cidx], cVb, semD)
            cp3.wait()
            cp4.wait()
        cp1.wait()
        cp2.wait()
        sidx[pl.ds(0, 16)] = slots

        def row_fn(i, carry2):
            sl = sidx[pl.ds(i, 16)][0]

            @pl.when(sl >= 0)
            def _():
                for cc in range(NCH):
                    rK[i, pl.ds(cc * 16, 16)] = cKb[i, pl.ds(cc * 16, 16)]
                    rV[i, pl.ds(cc * 16, 16)] = cVb[i, pl.ds(cc * 16, 16)]

            dec = dec_s[pl.ds(my + i, 16)][0]
            for j in range(N_H):
                bj = beta_l[pl.ds(t * 2 + j, 16)][0]
                kb = (t * 2 + j) * D
                a0k = jnp.zeros((16,), f32)
                a1k = jnp.zeros((16,), f32)
                a0v = jnp.zeros((16,), f32)
                a1v = jnp.zeros((16,), f32)
                for cc in range(0, NCH, 2):
                    kh0 = khat_l[pl.ds(kb + cc * 16, 16)]
                    kh1 = khat_l[pl.ds(kb + cc * 16 + 16, 16)]
                    a0k = a0k + kh0 * rK[i, pl.ds(cc * 16, 16)]
                    a1k = a1k + kh1 * rK[i, pl.ds(cc * 16 + 16, 16)]
                    a0v = a0v + kh0 * rV[i, pl.ds(cc * 16, 16)]
                    a1v = a1v + kh1 * rV[i, pl.ds(cc * 16 + 16, 16)]
                sk = jnp.sum(a0k + a1k)
                sv = jnp.sum(a0v + a1v)
                if j == 0:
                    dk = bj * (dec * sk)
                    dv = bj * (dec * sv)
                else:
                    dk = bj * sk
                    dv = bj * sv
                for cc in range(NCH):
                    kh = khat_l[pl.ds(kb + cc * 16, 16)]
                    ak = addk_l[pl.ds(kb + cc * 16, 16)]
                    av = addv_l[pl.ds(kb + cc * 16, 16)]
                    if j == 0:
                        rK[i, pl.ds(cc * 16, 16)] = (
                            rK[i, pl.ds(cc * 16, 16)] * dec - kh * dk + ak)
                        rV[i, pl.ds(cc * 16, 16)] = (
                            rV[i, pl.ds(cc * 16, 16)] * dec - kh * dv + av)
                    else:
                        rK[i, pl.ds(cc * 16, 16)] = (
                            rK[i, pl.ds(cc * 16, 16)] - kh * dk + ak)
                        rV[i, pl.ds(cc * 16, 16)] = (
                            rV[i, pl.ds(cc * 16, 16)] - kh * dv + av)
            return carry2
        lax.fori_loop(0, 16, row_fn, 0)
        pltpu.sync_copy(rK, cK_o.at[pl.ds(batch * 128 + my, 16)])
        pltpu.sync_copy(rV, cV_o.at[pl.ds(batch * 128 + my, 16)])
        for vh in range(2):
            wiv = widx_l[pl.ds(base + vh * 16, 16)]
            plsc.store_scatter(pos, [wiv], base + vh * 16 + ii16)
            dcv = dec_s[pl.ds(base + vh * 16, 16)]
            wvv = wv_s[pl.ds(base + vh * 16, 16)]
            zk = plsc.load_gather(zk_l, [wiv])
            plsc.store_scatter(zk_l, [wiv], dcv * zk + wvv)
            zv = plsc.load_gather(zv_l, [wiv])
            plsc.store_scatter(zv_l, [wiv], dcv * zv + wvv)
        plsc.subcore_barrier()
        ri = ridx_l[pl.ds(my, 16)]
        rsl = plsc.load_gather(pos, [ri])
        gidx2 = batch * M + ri
        cidx2 = batch * 128 + jnp.maximum(rsl, 0)
        ncached2 = plsc.all_reduce_population_count(rsl >= 0)[0]
        cp5 = pltpu.async_copy(Kf.at[gidx2], rK, semA)
        cp6 = pltpu.async_copy(Vf.at[gidx2], rV, semB)

        @pl.when(ncached2 > 0)
        def _():
            cp7 = pltpu.async_copy(cK_o.at[cidx2], cKb, semC)
            cp8 = pltpu.async_copy(cV_o.at[cidx2], cVb, semD)
            cp7.wait()
            cp8.wait()
        cp5.wait()
        cp6.wait()
        sidx[pl.ds(0, 16)] = rsl

        def sel_fn(i, carry2):
            sl = sidx[pl.ds(i, 16)][0]

            @pl.when(sl >= 0)
            def _():
                for cc in range(NCH):
                    rK[i, pl.ds(cc * 16, 16)] = cKb[i, pl.ds(cc * 16, 16)]
                    rV[i, pl.ds(cc * 16, 16)] = cVb[i, pl.ds(cc * 16, 16)]
            return carry2
        lax.fori_loop(0, 16, sel_fn, 0)
        szk[...] = plsc.load_gather(zk_l, [ri])
        szv[...] = plsc.load_gather(zv_l, [ri])

        def rel_fn(i, carry2):
            acc = jnp.zeros((16,), f32)
            for cc in range(NCH):
                acc = acc + (rK[i, pl.ds(cc * 16, 16)]
                             * h_l[pl.ds(t * D + cc * 16, 16)])
            return jnp.where(ii16 == i, jnp.sum(acc), carry2)
        relsum = lax.fori_loop(0, 16, rel_fn, jnp.zeros((16,), f32))
        srel[...] = relsum / szk[...] * scale
        pltpu.sync_copy(srel, relX.at[lb, half])
        plsc.subcore_barrier()
        pltpu.sync_copy(relX.at[lb, 1 - half], sorel)
        a = srel[...]
        bvec = sorel[...]
        m = jnp.maximum(jnp.max(a), jnp.max(bvec))
        ea = jnp.exp(a - m)
        eb = jnp.exp(bvec - m)
        ssum = jnp.sum(ea) + jnp.sum(eb)
        swv[...] = (ea / ssum) / szv[...]

        def ch_fn(cc, carry2):
            wv = swv[...]
            acc = jnp.zeros((16,), f32)
            for i in range(16):
                acc = acc + wv[i] * rV[i, pl.ds(cc * 16, 16)]
            part[pl.ds(cc * 16, 16)] = acc
            return carry2
        lax.fori_loop(0, NCH, ch_fn, 0)
        pltpu.sync_copy(part, partX.at[lb, half])
        plsc.subcore_barrier()

        @pl.when(half == 0)
        def _():
            pltpu.sync_copy(partX.at[lb, 1], tmp512)
            for cc in range(NCH):
                part[pl.ds(cc * 16, 16)] = (part[pl.ds(cc * 16, 16)]
                                            + tmp512[pl.ds(cc * 16, 16)])
            pltpu.sync_copy(part, ro_o.at[batch, pl.ds(t * D, D)])
        return carry
    lax.fori_loop(0, T, step, 0)

    @pl.when(half == 0)
    def _():
        pltpu.sync_copy(zk_l, zKo.at[batch])
        pltpu.sync_copy(zv_l, zVo.at[batch])


def _run_state(Kf, Vf, zK, zV, widx_f, ridx_f, dec_f, wv_f, khat_f,
               addk_f, addv_f, beta_f, h_f):
    f32 = jnp.float32
    i32 = jnp.int32
    mesh = plsc.VectorSubcoreMesh(core_axis_name="c", subcore_axis_name="s")
    outs = (jax.ShapeDtypeStruct((B, T * D), f32),
            jax.ShapeDtypeStruct((B, M), f32),
            jax.ShapeDtypeStruct((B, M), f32),
            jax.ShapeDtypeStruct((B * 128, D), f32),
            jax.ShapeDtypeStruct((B * 128, D), f32))
    scratch = [
        pltpu.VMEM((M,), i32),
        pltpu.VMEM((M,), f32),
        pltpu.VMEM((M,), f32),
        pltpu.VMEM((128,), i32),
        pltpu.VMEM((128,), i32),
        pltpu.VMEM((160,), f32),
        pltpu.VMEM((128,), f32),
        pltpu.VMEM((T * N_H * D,), f32),
        pltpu.VMEM((T * N_H * D,), f32),
        pltpu.VMEM((T * N_H * D,), f32),
        pltpu.VMEM((144,), f32),
        pltpu.VMEM((T * D,), f32),
        pltpu.VMEM((16, D), f32),
        pltpu.VMEM((16, D), f32),
        pltpu.VMEM((16, D), f32),
        pltpu.VMEM((16, D), f32),
        pltpu.VMEM((32,), i32),
        pltpu.VMEM((16,), f32),
        pltpu.VMEM((16,), f32),
        pltpu.VMEM((16,), f32),
        pltpu.VMEM((16,), f32),
        pltpu.VMEM((16,), f32),
        pltpu.VMEM((D,), f32),
        pltpu.VMEM((D,), f32),
        pltpu.VMEM_SHARED((8, 2, 16), f32),
        pltpu.VMEM_SHARED((8, 2, D), f32),
        pltpu.SemaphoreType.DMA,
        pltpu.SemaphoreType.DMA,
        pltpu.SemaphoreType.DMA,
        pltpu.SemaphoreType.DMA,
    ]
    fn = pl.kernel(_state_body, out_type=outs, mesh=mesh,
                   scratch_types=scratch,
                   compiler_params=pltpu.CompilerParams(
                       needs_layout_passes=False))
    return fn(Kf, Vf, zK, zV, widx_f, ridx_f, dec_f, wv_f, khat_f,
              addk_f, addv_f, beta_f, h_f)


BLKM = 2048
NBLK = M // BLKM


def _merge_body(keys_ref, kin, vin, ck, cv, kout, vout, p_ref):
    b = pl.program_id(0)
    n = pl.program_id(1)
    kout[...] = kin[...]
    vout[...] = vin[...]

    @pl.when(n == 0)
    def _():
        p_ref[0] = 0

    lim = (n + 1) * BLKM * 128

    def cond(p):
        return jnp.logical_and(p < 128, keys_ref[b, p] < lim)

    def body(p):
        key = keys_ref[b, p]
        idx = key >> 7
        slot = key & 127
        row = idx - n * BLKM
        kout[0, pl.ds(row, 1), :] = ck[0, pl.ds(slot, 1), :]
        vout[0, pl.ds(row, 1), :] = cv[0, pl.ds(slot, 1), :]
        return p + 1
    p_ref[0] = lax.while_loop(cond, body, p_ref[0])


def _run_merge(keys, K_slots, V_slots, cacheK, cacheV):
    f32 = jnp.float32
    grid_spec = pltpu.PrefetchScalarGridSpec(
        num_scalar_prefetch=1,
        grid=(B, NBLK),
        in_specs=[
            pl.BlockSpec((1, BLKM, D), lambda b, n, keys: (b, n, 0)),
            pl.BlockSpec((1, BLKM, D), lambda b, n, keys: (b, n, 0)),
            pl.BlockSpec((1, 128, D), lambda b, n, keys: (b, 0, 0)),
            pl.BlockSpec((1, 128, D), lambda b, n, keys: (b, 0, 0)),
        ],
        out_specs=[
            pl.BlockSpec((1, BLKM, D), lambda b, n, keys: (b, n, 0)),
            pl.BlockSpec((1, BLKM, D), lambda b, n, keys: (b, n, 0)),
        ],
        scratch_shapes=[pltpu.SMEM((1,), jnp.int32)],
    )
    return pl.pallas_call(
        _merge_body,
        grid_spec=grid_spec,
        out_shape=[jax.ShapeDtypeStruct((B, M, D), f32),
                   jax.ShapeDtypeStruct((B, M, D), f32)],
    )(keys, K_slots, V_slots, cacheK, cacheV)


def _post_body(ro_ref, gate_ref, wout_ref, y_ref):
    dn = (((1,), (1,)), ((), ()))
    y_ref[...] = gate_ref[...] * lax.dot_general(ro_ref[...], wout_ref[...], dn)


def _run_post(readout2, gate, W_out):
    return pl.pallas_call(
        _post_body,
        out_shape=jax.ShapeDtypeStruct((B * T, D), jnp.float32),
    )(readout2, gate, W_out)


def kernel(h, K_slots, V_slots, z_K, z_V, W_k, W_q, log_tau_w, log_tau_r,
           hh_k_w, hh_vk_w, hh_vv_w, hh_beta_w, hh_beta_b, gamma,
           W_out, W_gate):
    tau_w = jnp.exp(log_tau_w)
    tau_r = jnp.exp(log_tau_r)
    probs_list = []
    for t in range(T):
        h_t = h[:, t]
        zw = (h_t @ W_k.T).reshape(B, U, D_P)
        zr = (h_t @ W_q.T).reshape(B, U, D_P)
        probs_list.append(jax.nn.softmax(zw / tau_w, axis=-1))
        probs_list.append(jax.nn.softmax(zr / tau_r, axis=-1))
    pw = jnp.stack(probs_list[0::2], axis=1)
    pr = jnp.stack(probs_list[1::2], axis=1)
    probs = jnp.concatenate(
        [pw.reshape(B * T, U, D_P), pr.reshape(B * T, U, D_P)], axis=0)

    h2 = h.reshape(B * T, D)
    (oidx, oval, decay, khat, beta, addk, addv, gate) = _run_pre(
        probs, h2, hh_k_w, hh_vk_w, hh_vv_w,
        hh_beta_w.reshape(N_H, D), hh_beta_b.reshape(1, N_H),
        gamma, W_gate)

    widx = oidx[:B * T].reshape(B, T, TOPK)
    ridx = oidx[B * T:].reshape(B, T, TOPK)
    wval = oval[:B * T].reshape(B, T, TOPK)
    decay = decay.reshape(B, T, TOPK)
    khat = khat.reshape(N_H, B, T, D)
    beta = beta.reshape(B, T, N_H)
    addk = addk.reshape(N_H, B, T, D)
    addv = addv.reshape(N_H, B, T, D)

    i32 = jnp.int32
    widx_f = widx.reshape(B, T * TOPK)
    ridx_f = ridx.reshape(B, T * TOPK)
    dec_f = decay.reshape(B, T * TOPK)
    wv_f = wval.reshape(B, T * TOPK)
    khat_f = khat.transpose(1, 2, 0, 3).reshape(B, T * N_H * D)
    addk_f = addk.transpose(1, 2, 0, 3).reshape(B, T * N_H * D)
    addv_f = addv.transpose(1, 2, 0, 3).reshape(B, T * N_H * D)
    beta_f = jnp.concatenate(
        [beta.reshape(B, T * N_H), jnp.zeros((B, 120), jnp.float32)], axis=1)
    h_f = h.reshape(B, T * D)
    ro, zKo, zVo, cKf, cVf = _run_state(
        K_slots.reshape(B * M, D), V_slots.reshape(B * M, D), z_K, z_V,
        widx_f, ridx_f, dec_f, wv_f, khat_f, addk_f, addv_f, beta_f, h_f)

    keys = jnp.sort(widx_f * 128
                    + jnp.arange(T * TOPK, dtype=i32)[None, :], axis=1)
    K_out, V_out = _run_merge(keys, K_slots, V_slots,
                              cKf.reshape(B, 128, D), cVf.reshape(B, 128, D))

    y = _run_post(ro.reshape(B * T, D), gate, W_out).reshape(B, T, D)
    return (y, K_out, V_out, zKo, zVo, widx[:, T - 1], ridx[:, T - 1])
